# baseline (speedup 1.0000x reference)
import jax
import jax.numpy as jnp
from jax import lax
from jax.experimental import pallas as pl
from jax.experimental.pallas import tpu as pltpu
from jax.experimental.pallas import tpu_sc as plsc

B = 16
FRAMES = 32768
FLEN = 16
OUT_LEN = 2 * (FRAMES - 1) + FLEN
F = 2048
CPB = FRAMES // F
CPW = CPB // 2
NT = 18
KTSZ = 256 * 8 * 128
IN_SZ = 2 * NT * 1024
OT = 513
NBUF = 2


def _oa_body(s5_hbm, o4_hbm, in0, in1, ob0, ob1, is0, is1, os0, os1):
    wid = lax.axis_index("s") * 2 + lax.axis_index("c")
    batch = wid // 2
    half = wid % 2
    bt = batch // 8
    bp = batch % 8

    in_bufs = (in0, in1)
    out_bufs = (ob0, ob1)
    in_sems = (is0, is1)
    out_sems = (os0, os1)

    iota = lax.iota(jnp.int32, 16)
    phi = (iota >> 1) + 128 * (iota & 1)
    phi_j = [phi] + [phi + jnp.where((iota >> 1) < j, -896, 0)
                     for j in range(1, 8)]
    zvec = jnp.zeros((16,), jnp.float32)

    def chunk_f0(i):
        return (half * CPW + i) * F

    def issue_in(i):
        f0 = chunk_f0(i)
        nb = i % NBUF
        descs = []
        if i == 0:
            for kt in range(2):
                src = (batch * 2 + kt) * KTSZ
                descs.append(pltpu.async_copy(
                    s5_hbm.at[pl.ds(src + (f0 // 128) * 1024, 16 * 1024)],
                    in_bufs[nb].at[pl.ds(kt * NT * 1024 + 1024, 16 * 1024)],
                    in_sems[nb]))
                descs.append(pltpu.async_copy(
                    s5_hbm.at[pl.ds(src + lax.max(f0 // 128 - 1, 0) * 1024,
                                    1024)],
                    in_bufs[nb].at[pl.ds(kt * NT * 1024, 1024)],
                    in_sems[nb]))
        else:
            for kt in range(2):
                src = (batch * 2 + kt) * KTSZ + (f0 // 128 - 1) * 1024
                descs.append(pltpu.async_copy(
                    s5_hbm.at[pl.ds(src, 17 * 1024)],
                    in_bufs[nb].at[pl.ds(kt * NT * 1024, 17 * 1024)],
                    in_sems[nb]))
        return descs

    def compute_group(buf, rp_plus1, rhalf, c0):
        fpb = 64 * rhalf + c0 // 2
        acc = None
        for j in range(8):
            const = 17408 * (j >= 4) + 256 * j + fpb - j
            vec = phi_j[j] if fpb == 0 else phi
            g = plsc.load_gather(buf, [vec + (1024 * rp_plus1 + const)])
            acc = g if acc is None else acc + g
        return acc

    in_descs = [issue_in(i) for i in range(NBUF)]

    zb = in_bufs[(CPW - 1) % NBUF]
    for kt in range(2):
        for v in range(64):
            zb[pl.ds(kt * NT * 1024 + 17 * 1024 + 16 * v, 16)] = zvec

    out_descs = [None] * CPW

    for i in range(CPW):
        nb = i % NBUF
        buf, ob = in_bufs[nb], out_bufs[i % 2]
        for d in in_descs[i]:
            d.wait()

        if i == 0:
            @pl.when(half == 0)
            def _():
                for kt in range(2):
                    for v in range(64):
                        buf[pl.ds(kt * NT * 1024 + 16 * v, 16)] = zvec

        if i >= 2:
            out_descs[i - 2].wait()

        @plsc.parallel_loop(0, 16)
        def _(rp):
            for rhalf in range(2):
                r = 2 * rp + rhalf
                for c0 in range(0, 128, 16):
                    ob[r, pl.ds(c0, 16)] = compute_group(
                        buf, rp + 1, rhalf, c0)

        if i == CPW - 1:
            @pl.when(half == 1)
            def _():
                for c0 in range(0, 128, 16):
                    ob[32, pl.ds(c0, 16)] = compute_group(buf, 17, 0, c0)

        cib = half * CPW + i
        out_descs[i] = pltpu.async_copy(
            ob.at[pl.ds(0, 32), :],
            o4_hbm.at[bt, pl.ds(32 * cib, 32), bp, :],
            out_sems[i % 2])
        if i == CPW - 1:
            @pl.when(half == 1)
            def _():
                pltpu.sync_copy(ob.at[32, :], o4_hbm.at[bt, 512, bp, :])

        if i + NBUF < CPW:
            in_descs.append(issue_in(i + NBUF))

    out_descs[CPW - 2].wait()
    out_descs[CPW - 1].wait()


_oa_kernel = pl.kernel(
    _oa_body,
    out_type=jax.ShapeDtypeStruct((2, OT, 8, 128), jnp.float32),
    mesh=plsc.VectorSubcoreMesh(core_axis_name="c", subcore_axis_name="s"),
    scratch_types=[
        pltpu.VMEM((IN_SZ,), jnp.float32),
        pltpu.VMEM((IN_SZ,), jnp.float32),
        pltpu.VMEM((33, 128), jnp.float32),
        pltpu.VMEM((33, 128), jnp.float32),
        pltpu.SemaphoreType.DMA,
        pltpu.SemaphoreType.DMA,
        pltpu.SemaphoreType.DMA,
        pltpu.SemaphoreType.DMA,
    ],
    compiler_params=pltpu.CompilerParams(needs_layout_passes=False,
                                         use_tc_tiling_on_sc=False,
                                         skip_device_barrier=True),
)


@jax.jit
def kernel(signal):
    s5 = signal.reshape(B, 256, 128, 2, 8).transpose(0, 3, 1, 4, 2)
    o4 = _oa_kernel(s5.reshape(-1))
    return o4.transpose(0, 2, 1, 3).reshape(B, OT * 128)[:, :OUT_LEN]


# speedup vs baseline: 1.0498x; 1.0498x over previous
import jax
import jax.numpy as jnp
from jax import lax
from jax.experimental import pallas as pl
from jax.experimental.pallas import tpu as pltpu
from jax.experimental.pallas import tpu_sc as plsc

B = 16
FRAMES = 32768
FLEN = 16
OUT_LEN = 2 * (FRAMES - 1) + FLEN
F = 2048
CPB = FRAMES // F
CPW = CPB // 2
NT = 18
KTSZ = 256 * 8 * 128
IN_SZ = 2 * NT * 1024
OT = 513
NBUF = 2


def _oa_body(s5_hbm, o4_hbm, in0, in1, ob0, ob1, is0, is1, os0, os1):
    wid = lax.axis_index("s") * 2 + lax.axis_index("c")
    batch = wid // 2
    half = wid % 2
    bt = batch // 8
    bp = batch % 8

    in_bufs = (in0, in1)
    out_bufs = (ob0, ob1)
    in_sems = (is0, is1)
    out_sems = (os0, os1)

    iota = lax.iota(jnp.int32, 16)
    vec_j = [iota] + [iota + jnp.where(iota < j, -896, 0)
                      for j in range(1, 8)]
    col2 = 2 * iota
    zvec = jnp.zeros((16,), jnp.float32)

    def chunk_f0(i):
        return (half * CPW + i) * F

    def issue_in(i):
        f0 = chunk_f0(i)
        nb = i % NBUF
        descs = []
        if i == 0:
            for kt in range(2):
                src = (batch * 2 + kt) * KTSZ
                descs.append(pltpu.async_copy(
                    s5_hbm.at[pl.ds(src + (f0 // 128) * 1024, 16 * 1024)],
                    in_bufs[nb].at[pl.ds(kt * NT * 1024 + 1024, 16 * 1024)],
                    in_sems[nb]))
                descs.append(pltpu.async_copy(
                    s5_hbm.at[pl.ds(src + lax.max(f0 // 128 - 1, 0) * 1024,
                                    1024)],
                    in_bufs[nb].at[pl.ds(kt * NT * 1024, 1024)],
                    in_sems[nb]))
        else:
            for kt in range(2):
                src = (batch * 2 + kt) * KTSZ + (f0 // 128 - 1) * 1024
                descs.append(pltpu.async_copy(
                    s5_hbm.at[pl.ds(src, 17 * 1024)],
                    in_bufs[nb].at[pl.ds(kt * NT * 1024, 17 * 1024)],
                    in_sems[nb]))
        return descs

    def store_row(buf, ob, r_dyn, rp_plus1, rhalf):
        rvec = jnp.full((16,), r_dyn, jnp.int32)
        for c in (0, 16, 32, 48):
            for p in range(2):
                acc = None
                for j in range(8):
                    k = 2 * j + p
                    const = ((k // 8) * 18432 + (k % 8) * 128
                             + 64 * rhalf + c - j)
                    vec = vec_j[j] if (rhalf == 0 and c == 0) else vec_j[0]
                    g = plsc.load_gather(
                        buf, [vec + (1024 * rp_plus1 + const)])
                    acc = g if acc is None else acc + g
                plsc.store_scatter(ob, [rvec, col2 + (2 * c + p)], acc)

    in_descs = [issue_in(i) for i in range(NBUF)]

    zb = in_bufs[(CPW - 1) % NBUF]
    for kt in range(2):
        for v in range(64):
            zb[pl.ds(kt * NT * 1024 + 17 * 1024 + 16 * v, 16)] = zvec

    out_descs = [None] * CPW

    for i in range(CPW):
        nb = i % NBUF
        buf, ob = in_bufs[nb], out_bufs[i % 2]
        for d in in_descs[i]:
            d.wait()

        if i == 0:
            @pl.when(half == 0)
            def _():
                for kt in range(2):
                    for v in range(64):
                        buf[pl.ds(kt * NT * 1024 + 16 * v, 16)] = zvec

        if i >= 2:
            out_descs[i - 2].wait()

        @plsc.parallel_loop(0, 16)
        def _(rp):
            for rhalf in range(2):
                store_row(buf, ob, 2 * rp + rhalf, rp + 1, rhalf)

        if i == CPW - 1:
            @pl.when(half == 1)
            def _():
                store_row(buf, ob, 32, 17, 0)

        cib = half * CPW + i
        out_descs[i] = pltpu.async_copy(
            ob.at[pl.ds(0, 32), :],
            o4_hbm.at[bt, pl.ds(32 * cib, 32), bp, :],
            out_sems[i % 2])
        if i == CPW - 1:
            @pl.when(half == 1)
            def _():
                pltpu.sync_copy(ob.at[32, :], o4_hbm.at[bt, 512, bp, :])

        if i + NBUF < CPW:
            in_descs.append(issue_in(i + NBUF))

    out_descs[CPW - 2].wait()
    out_descs[CPW - 1].wait()


_oa_kernel = pl.kernel(
    _oa_body,
    out_type=jax.ShapeDtypeStruct((2, OT, 8, 128), jnp.float32),
    mesh=plsc.VectorSubcoreMesh(core_axis_name="c", subcore_axis_name="s"),
    scratch_types=[
        pltpu.VMEM((IN_SZ,), jnp.float32),
        pltpu.VMEM((IN_SZ,), jnp.float32),
        pltpu.VMEM((33, 128), jnp.float32),
        pltpu.VMEM((33, 128), jnp.float32),
        pltpu.SemaphoreType.DMA,
        pltpu.SemaphoreType.DMA,
        pltpu.SemaphoreType.DMA,
        pltpu.SemaphoreType.DMA,
    ],
    compiler_params=pltpu.CompilerParams(needs_layout_passes=False,
                                         use_tc_tiling_on_sc=False),
)


@jax.jit
def kernel(signal):
    s5 = signal.reshape(B, 256, 128, 2, 8).transpose(0, 3, 1, 4, 2)
    o4 = _oa_kernel(s5.reshape(-1))
    return o4.transpose(0, 2, 1, 3).reshape(B, OT * 128)[:, :OUT_LEN]


# speedup vs baseline: 1.1341x; 1.0803x over previous
import jax
import jax.numpy as jnp
from jax import lax
from jax.experimental import pallas as pl
from jax.experimental.pallas import tpu as pltpu
from jax.experimental.pallas import tpu_sc as plsc

B = 16
FRAMES = 32768
FLEN = 16
OUT_LEN = 2 * (FRAMES - 1) + FLEN
F = 2048
CPB = FRAMES // F
CPW = CPB // 2
NT = 18
KTSZ = 256 * 8 * 128
IN_SZ = 2 * NT * 1024
OT = 513
NBUF = 2


def _oa_body(s5_hbm, o4_hbm, in0, in1, ob0, ob1, is0, is1, os0, os1):
    wid = lax.axis_index("s") * 2 + lax.axis_index("c")
    batch = wid // 2
    half = wid % 2
    bt = batch // 8
    bp = batch % 8

    in_bufs = (in0, in1)
    out_bufs = (ob0, ob1)
    in_sems = (is0, is1)
    out_sems = (os0, os1)

    iota = lax.iota(jnp.int32, 16)
    vec_j = [iota] + [iota + jnp.where(iota < j, -896, 0)
                      for j in range(1, 8)]
    col2 = 2 * iota
    zvec = jnp.zeros((16,), jnp.float32)

    def chunk_f0(i):
        return (half * CPW + i) * F

    def issue_in(i):
        f0 = chunk_f0(i)
        nb = i % NBUF
        descs = []
        if i == 0:
            for kt in range(2):
                src = (batch * 2 + kt) * KTSZ
                descs.append(pltpu.async_copy(
                    s5_hbm.at[pl.ds(src + (f0 // 128) * 1024, 16 * 1024)],
                    in_bufs[nb].at[pl.ds(kt * NT * 1024 + 1024, 16 * 1024)],
                    in_sems[nb]))
                descs.append(pltpu.async_copy(
                    s5_hbm.at[pl.ds(src + lax.max(f0 // 128 - 1, 0) * 1024,
                                    1024)],
                    in_bufs[nb].at[pl.ds(kt * NT * 1024, 1024)],
                    in_sems[nb]))
        else:
            for kt in range(2):
                src = (batch * 2 + kt) * KTSZ + (f0 // 128 - 1) * 1024
                descs.append(pltpu.async_copy(
                    s5_hbm.at[pl.ds(src, 17 * 1024)],
                    in_bufs[nb].at[pl.ds(kt * NT * 1024, 17 * 1024)],
                    in_sems[nb]))
        return descs

    def store_row(buf, ob, r_dyn, rp_plus1, rhalf):
        rvec = jnp.full((16,), r_dyn, jnp.int32)
        for c in (0, 16, 32, 48):
            for p in range(2):
                acc = None
                for j in range(8):
                    k = 2 * j + p
                    const = ((k // 8) * 18432 + (k % 8) * 128
                             + 64 * rhalf + c - j)
                    vec = vec_j[j] if (rhalf == 0 and c == 0) else vec_j[0]
                    g = plsc.load_gather(
                        buf, [vec + (1024 * rp_plus1 + const)])
                    acc = g if acc is None else acc + g
                plsc.store_scatter(ob, [rvec, col2 + (2 * c + p)], acc)

    in_descs = [issue_in(i) for i in range(NBUF)]

    zb = in_bufs[(CPW - 1) % NBUF]
    for kt in range(2):
        for v in range(64):
            zb[pl.ds(kt * NT * 1024 + 17 * 1024 + 16 * v, 16)] = zvec

    out_descs = [None] * CPW

    for i in range(CPW):
        nb = i % NBUF
        buf, ob = in_bufs[nb], out_bufs[i % 2]
        for d in in_descs[i]:
            d.wait()

        if i == 0:
            @pl.when(half == 0)
            def _():
                for kt in range(2):
                    for v in range(64):
                        buf[pl.ds(kt * NT * 1024 + 16 * v, 16)] = zvec

        if i >= 2:
            out_descs[i - 2].wait()

        @plsc.parallel_loop(0, 16, unroll=2)
        def _(rp):
            for rhalf in range(2):
                store_row(buf, ob, 2 * rp + rhalf, rp + 1, rhalf)

        if i == CPW - 1:
            @pl.when(half == 1)
            def _():
                store_row(buf, ob, 32, 17, 0)

        cib = half * CPW + i
        out_descs[i] = pltpu.async_copy(
            ob.at[pl.ds(0, 32), :],
            o4_hbm.at[bt, pl.ds(32 * cib, 32), bp, :],
            out_sems[i % 2])
        if i == CPW - 1:
            @pl.when(half == 1)
            def _():
                pltpu.sync_copy(ob.at[32, :], o4_hbm.at[bt, 512, bp, :])

        if i + NBUF < CPW:
            in_descs.append(issue_in(i + NBUF))

    out_descs[CPW - 2].wait()
    out_descs[CPW - 1].wait()


_oa_kernel = pl.kernel(
    _oa_body,
    out_type=jax.ShapeDtypeStruct((2, OT, 8, 128), jnp.float32),
    mesh=plsc.VectorSubcoreMesh(core_axis_name="c", subcore_axis_name="s"),
    scratch_types=[
        pltpu.VMEM((IN_SZ,), jnp.float32),
        pltpu.VMEM((IN_SZ,), jnp.float32),
        pltpu.VMEM((33, 128), jnp.float32),
        pltpu.VMEM((33, 128), jnp.float32),
        pltpu.SemaphoreType.DMA,
        pltpu.SemaphoreType.DMA,
        pltpu.SemaphoreType.DMA,
        pltpu.SemaphoreType.DMA,
    ],
    compiler_params=pltpu.CompilerParams(needs_layout_passes=False,
                                         use_tc_tiling_on_sc=False),
)


@jax.jit
def kernel(signal):
    s5 = signal.reshape(B, 256, 128, 2, 8).transpose(0, 3, 1, 4, 2)
    o4 = _oa_kernel(s5.reshape(-1))
    return o4.transpose(0, 2, 1, 3).reshape(B, OT * 128)[:, :OUT_LEN]
